# initial kernel scaffold (unmeasured)
import functools

import jax
import jax.numpy as jnp
from jax import lax
from jax.experimental import pallas as pl
from jax.experimental.pallas import tpu as pltpu

N_DEV = 4


def kernel(x, W1, W2):
    M, _ = x.shape
    _, D = W1.shape
    _, N = W2.shape
    CH = M // N_DEV

    def body(x_ref, w1_ref, w2_ref, out_ref,
             hp_ref, hf_ref, acc_ref, rs_buf,
             rs_send_sems, rs_recv_sems, ag_send_sems, ag_recv_sems):
        me = lax.axis_index("i")
        left = lax.rem(me + N_DEV - 1, N_DEV)
        right = lax.rem(me + 1, N_DEV)

        barrier = pltpu.get_barrier_semaphore()
        for nbr in (left, right):
            pl.semaphore_signal(barrier, inc=1, device_id=(nbr,),
                                device_id_type=pl.DeviceIdType.MESH)
        pl.semaphore_wait(barrier, 2)

        for c in range(N_DEV):
            hp_ref[c] = jnp.dot(x_ref[c * CH:(c + 1) * CH, :], w1_ref[...],
                                preferred_element_type=jnp.float32)

        for s in range(N_DEV - 1):
            src = hp_ref.at[me] if s == 0 else acc_ref
            rdma = pltpu.make_async_remote_copy(
                src_ref=src,
                dst_ref=rs_buf.at[s],
                send_sem=rs_send_sems.at[s],
                recv_sem=rs_recv_sems.at[s],
                device_id=(right,),
                device_id_type=pl.DeviceIdType.MESH,
            )
            rdma.start()
            rdma.wait()
            c_recv = lax.rem(me + N_DEV - 1 - s, N_DEV)
            acc_ref[...] = rs_buf[s] + hp_ref[c_recv]

        own = lax.rem(me + 1, N_DEV)
        hf_ref[own] = acc_ref[...]

        for t in range(N_DEV - 1):
            send_c = lax.rem(me + 1 - t + N_DEV, N_DEV)
            rdma = pltpu.make_async_remote_copy(
                src_ref=hf_ref.at[send_c],
                dst_ref=hf_ref.at[send_c],
                send_sem=ag_send_sems.at[t],
                recv_sem=ag_recv_sems.at[t],
                device_id=(right,),
                device_id_type=pl.DeviceIdType.MESH,
            )
            rdma.start()
            rdma.wait()

        for c in range(N_DEV):
            out_ref[c * CH:(c + 1) * CH, :] = jnp.dot(
                hf_ref[c], w2_ref[...], preferred_element_type=jnp.float32)

        @functools.partial(pl.run_scoped, sem=pltpu.SemaphoreType.REGULAR)
        def _(sem):
            for nbr in (left, right):
                pl.semaphore_signal(sem, inc=1, device_id=(nbr,),
                                    device_id_type=pl.DeviceIdType.MESH)
            pl.semaphore_wait(sem, 2)

    return pl.pallas_call(
        body,
        out_shape=jax.ShapeDtypeStruct((M, N), jnp.float32),
        in_specs=[pl.BlockSpec(memory_space=pltpu.VMEM)] * 3,
        out_specs=pl.BlockSpec(memory_space=pltpu.VMEM),
        scratch_shapes=[
            pltpu.VMEM((N_DEV, CH, D), jnp.float32),
            pltpu.VMEM((N_DEV, CH, D), jnp.float32),
            pltpu.VMEM((CH, D), jnp.float32),
            pltpu.VMEM((N_DEV - 1, CH, D), jnp.float32),
            pltpu.SemaphoreType.DMA((N_DEV - 1,)),
            pltpu.SemaphoreType.DMA((N_DEV - 1,)),
            pltpu.SemaphoreType.DMA((N_DEV - 1,)),
            pltpu.SemaphoreType.DMA((N_DEV - 1,)),
        ],
        compiler_params=pltpu.CompilerParams(collective_id=0),
    )(x, W1, W2)


# baseline (device time: 331267 ns/iter reference)
import functools

import jax
import jax.numpy as jnp
from jax import lax
from jax.experimental import pallas as pl
from jax.experimental.pallas import tpu as pltpu

N_DEV = 4


def kernel(x, W1, W2):
    M, _ = x.shape
    _, D = W1.shape
    _, N = W2.shape
    CH = M // N_DEV

    def body(x_ref, w1_ref, w2_ref, out_ref, h_ref,
             rs_send_sems, rs_recv_sems, ag_send_sems, ag_recv_sems):
        rs_slot = lambda s: out_ref.at[pl.ds(s * CH, CH), :]
        me = lax.axis_index("i")
        left = lax.rem(me + N_DEV - 1, N_DEV)
        right = lax.rem(me + 1, N_DEV)

        barrier = pltpu.get_barrier_semaphore()
        for nbr in (left, right):
            pl.semaphore_signal(barrier, inc=1, device_id=(nbr,),
                                device_id_type=pl.DeviceIdType.MESH)
        pl.semaphore_wait(barrier, 2)

        for c in range(N_DEV):
            h_ref[c] = jnp.dot(x_ref[c * CH:(c + 1) * CH, :], w1_ref[...],
                               preferred_element_type=jnp.float32)

        for s in range(N_DEV - 1):
            src = h_ref.at[me] if s == 0 else rs_slot(s - 1)
            rdma = pltpu.make_async_remote_copy(
                src_ref=src,
                dst_ref=rs_slot(s),
                send_sem=rs_send_sems.at[s],
                recv_sem=rs_recv_sems.at[s],
                device_id=(right,),
                device_id_type=pl.DeviceIdType.MESH,
            )
            rdma.start()
            rdma.wait()
            c_recv = lax.rem(me + N_DEV - 1 - s, N_DEV)
            sl = pl.ds(s * CH, CH)
            out_ref[sl, :] = out_ref[sl, :] + h_ref[c_recv]

        own = lax.rem(me + 1, N_DEV)
        h_ref[own] = out_ref[(N_DEV - 2) * CH:(N_DEV - 1) * CH, :]

        for t in range(N_DEV - 1):
            send_c = lax.rem(me + 1 - t + N_DEV, N_DEV)
            rdma = pltpu.make_async_remote_copy(
                src_ref=h_ref.at[send_c],
                dst_ref=h_ref.at[send_c],
                send_sem=ag_send_sems.at[t],
                recv_sem=ag_recv_sems.at[t],
                device_id=(right,),
                device_id_type=pl.DeviceIdType.MESH,
            )
            rdma.start()
            rdma.wait()

        for c in range(N_DEV):
            out_ref[c * CH:(c + 1) * CH, :] = jnp.dot(
                h_ref[c], w2_ref[...], preferred_element_type=jnp.float32)

        @functools.partial(pl.run_scoped, sem=pltpu.SemaphoreType.REGULAR)
        def _(sem):
            for nbr in (left, right):
                pl.semaphore_signal(sem, inc=1, device_id=(nbr,),
                                    device_id_type=pl.DeviceIdType.MESH)
            pl.semaphore_wait(sem, 2)

    return pl.pallas_call(
        body,
        out_shape=jax.ShapeDtypeStruct((M, N), jnp.float32),
        in_specs=[pl.BlockSpec(memory_space=pltpu.VMEM)] * 3,
        out_specs=pl.BlockSpec(memory_space=pltpu.VMEM),
        scratch_shapes=[
            pltpu.VMEM((N_DEV, CH, D), jnp.float32),
            pltpu.SemaphoreType.DMA((N_DEV - 1,)),
            pltpu.SemaphoreType.DMA((N_DEV - 1,)),
            pltpu.SemaphoreType.DMA((N_DEV - 1,)),
            pltpu.SemaphoreType.DMA((N_DEV - 1,)),
        ],
        compiler_params=pltpu.CompilerParams(
            collective_id=0, vmem_limit_bytes=63 * 1024 * 1024),
    )(x, W1, W2)


# device time: 182569 ns/iter; 1.8145x vs baseline; 1.8145x over previous
import functools

import jax
import jax.numpy as jnp
from jax import lax
from jax.experimental import pallas as pl
from jax.experimental.pallas import tpu as pltpu

N_DEV = 4


def kernel(x, W1, W2):
    M, _ = x.shape
    _, D = W1.shape
    _, N = W2.shape
    CH = M // N_DEV
    HALF = CH // 2

    def body(x_ref, w1_ref, w2_ref, out_ref, h_ref,
             rs_r_send, rs_r_recv, rs_l_send, rs_l_recv,
             ag_r_send, ag_r_recv, ag_l_send, ag_l_recv):
        me = lax.axis_index("i")
        left = lax.rem(me + N_DEV - 1, N_DEV)
        right = lax.rem(me + 1, N_DEV)
        md = lambda c: lax.rem(c + 2 * N_DEV, N_DEV)

        top = lambda c: h_ref.at[2 * md(c)]
        bot = lambda c: h_ref.at[2 * md(c) + 1]
        slot_t = lambda s: out_ref.at[pl.ds(s * CH, HALF), :]
        slot_b = lambda s: out_ref.at[pl.ds(s * CH + HALF, HALF), :]

        def gemm1(c):
            cc = md(c)
            h_ref[2 * cc] = jnp.dot(
                x_ref[pl.ds(cc * CH, HALF), :], w1_ref[...],
                preferred_element_type=jnp.float32)
            h_ref[2 * cc + 1] = jnp.dot(
                x_ref[pl.ds(cc * CH + HALF, HALF), :], w1_ref[...],
                preferred_element_type=jnp.float32)

        def gemm2(c):
            cc = md(c)
            out_ref[pl.ds(cc * CH, HALF), :] = jnp.dot(
                h_ref[2 * cc], w2_ref[...],
                preferred_element_type=jnp.float32)
            out_ref[pl.ds(cc * CH + HALF, HALF), :] = jnp.dot(
                h_ref[2 * cc + 1], w2_ref[...],
                preferred_element_type=jnp.float32)

        barrier = pltpu.get_barrier_semaphore()
        for nbr in (left, right):
            pl.semaphore_signal(barrier, inc=1, device_id=(nbr,),
                                device_id_type=pl.DeviceIdType.MESH)
        pl.semaphore_wait(barrier, 2)

        gemm1(me)

        for s in range(N_DEV - 1):
            r_r = pltpu.make_async_remote_copy(
                src_ref=top(me) if s == 0 else slot_t(s - 1),
                dst_ref=slot_t(s),
                send_sem=rs_r_send.at[s], recv_sem=rs_r_recv.at[s],
                device_id=(right,), device_id_type=pl.DeviceIdType.MESH,
            )
            r_l = pltpu.make_async_remote_copy(
                src_ref=bot(me) if s == 0 else slot_b(s - 1),
                dst_ref=slot_b(s),
                send_sem=rs_l_send.at[s], recv_sem=rs_l_recv.at[s],
                device_id=(left,), device_id_type=pl.DeviceIdType.MESH,
            )
            r_r.start()
            r_l.start()
            if s == 0:
                gemm1(me - 1)
                gemm1(me + 1)
            elif s == 1:
                gemm1(me + 2)
            r_r.wait()
            out_ref[pl.ds(s * CH, HALF), :] = (
                out_ref[pl.ds(s * CH, HALF), :] + h_ref[2 * md(me - 1 - s)])
            r_l.wait()
            out_ref[pl.ds(s * CH + HALF, HALF), :] = (
                out_ref[pl.ds(s * CH + HALF, HALF), :]
                + h_ref[2 * md(me + 1 + s) + 1])

        h_ref[2 * md(me + 1)] = out_ref[pl.ds((N_DEV - 2) * CH, HALF), :]
        h_ref[2 * md(me - 1) + 1] = out_ref[
            pl.ds((N_DEV - 2) * CH + HALF, HALF), :]

        for t in range(N_DEV - 1):
            a_r = pltpu.make_async_remote_copy(
                src_ref=top(me + 1 - t), dst_ref=top(me + 1 - t),
                send_sem=ag_r_send.at[t], recv_sem=ag_r_recv.at[t],
                device_id=(right,), device_id_type=pl.DeviceIdType.MESH,
            )
            a_l = pltpu.make_async_remote_copy(
                src_ref=bot(me - 1 + t), dst_ref=bot(me - 1 + t),
                send_sem=ag_l_send.at[t], recv_sem=ag_l_recv.at[t],
                device_id=(left,), device_id_type=pl.DeviceIdType.MESH,
            )
            a_r.start()
            a_l.start()
            if t == 1:
                gemm2(me)
            elif t == 2:
                gemm2(me + 1)
                gemm2(me - 1)
            a_r.wait()
            a_l.wait()
        gemm2(me + 2)

        @functools.partial(pl.run_scoped, sem=pltpu.SemaphoreType.REGULAR)
        def _(sem):
            for nbr in (left, right):
                pl.semaphore_signal(sem, inc=1, device_id=(nbr,),
                                    device_id_type=pl.DeviceIdType.MESH)
            pl.semaphore_wait(sem, 2)

    return pl.pallas_call(
        body,
        out_shape=jax.ShapeDtypeStruct((M, N), jnp.float32),
        in_specs=[pl.BlockSpec(memory_space=pltpu.VMEM)] * 3,
        out_specs=pl.BlockSpec(memory_space=pltpu.VMEM),
        scratch_shapes=[
            pltpu.VMEM((2 * N_DEV, HALF, D), jnp.float32),
            pltpu.SemaphoreType.DMA((N_DEV - 1,)),
            pltpu.SemaphoreType.DMA((N_DEV - 1,)),
            pltpu.SemaphoreType.DMA((N_DEV - 1,)),
            pltpu.SemaphoreType.DMA((N_DEV - 1,)),
            pltpu.SemaphoreType.DMA((N_DEV - 1,)),
            pltpu.SemaphoreType.DMA((N_DEV - 1,)),
            pltpu.SemaphoreType.DMA((N_DEV - 1,)),
            pltpu.SemaphoreType.DMA((N_DEV - 1,)),
        ],
        compiler_params=pltpu.CompilerParams(
            collective_id=0, vmem_limit_bytes=63 * 1024 * 1024),
    )(x, W1, W2)


# device time: 102494 ns/iter; 3.2321x vs baseline; 1.7813x over previous
import functools
import os

import jax
import jax.numpy as jnp
from jax import lax
from jax.experimental import pallas as pl
from jax.experimental.pallas import tpu as pltpu

N_DEV = 4
NQ = int(os.environ.get("KERNEL_NQ", "2"))
COMM_ONLY = os.environ.get("KERNEL_COMM_ONLY", "0") == "1"


def kernel(x, W1, W2):
    M, _ = x.shape
    _, D = W1.shape
    _, N = W2.shape
    CH = M // N_DEV
    HALF = CH // 2
    QR = HALF // NQ

    STREAMS = tuple((d, q) for q in range(NQ) for d in (0, 1))

    def body(x_ref, w1_ref, w2_ref, out_ref, h_ref, rs_buf, w2b_ref,
             rs_send0, rs_recv0, rs_send1, rs_recv1,
             ag_send0, ag_recv0, ag_send1, ag_recv1):
        me = lax.axis_index("i")
        left = lax.rem(me + N_DEV - 1, N_DEV)
        right = lax.rem(me + 1, N_DEV)
        md = lambda c: lax.rem(c + 2 * N_DEV, N_DEV)

        rs_sems = {0: (rs_send0, rs_recv0), 1: (rs_send1, rs_recv1)}
        ag_sems = {0: (ag_send0, ag_recv0), 1: (ag_send1, ag_recv1)}
        peer = {0: right, 1: left}

        def h_slot(c, d, q):
            return 2 * NQ * md(c) + NQ * d + q

        def row_off(cc, d, q):
            return cc * CH + d * HALF + q * QR

        def rs_region(s, d, q):
            return pl.ds(s * CH + d * HALF + q * QR, QR)

        def gemm1_piece(c, d, q):
            cc = md(c)
            if COMM_ONLY:
                h_ref[h_slot(c, d, q)] = x_ref[
                    pl.ds(row_off(cc, d, q), QR), :].astype(jnp.bfloat16)
            else:
                h_ref[h_slot(c, d, q)] = jnp.dot(
                    x_ref[pl.ds(row_off(cc, d, q), QR), :], w1_ref[...],
                    preferred_element_type=jnp.float32).astype(jnp.bfloat16)

        def gemm1(c):
            for d, q in STREAMS:
                gemm1_piece(c, d, q)

        def gemm2_piece(c, d, q):
            cc = md(c)
            if COMM_ONLY:
                out_ref[pl.ds(row_off(cc, d, q), QR), :] = h_ref[
                    h_slot(c, d, q)].astype(jnp.float32)
            else:
                out_ref[pl.ds(row_off(cc, d, q), QR), :] = jnp.dot(
                    h_ref[h_slot(c, d, q)], w2b_ref[...],
                    preferred_element_type=jnp.float32)

        def gemm2(c):
            for d, q in STREAMS:
                gemm2_piece(c, d, q)

        def rs_send(s, d, q):
            send_sems, recv_sems = rs_sems[d]
            rdma = pltpu.make_async_remote_copy(
                src_ref=(h_ref.at[h_slot(me, d, q)] if s == 0
                         else rs_buf.at[rs_region(s - 1, d, q), :]),
                dst_ref=rs_buf.at[rs_region(s, d, q), :],
                send_sem=send_sems.at[s, q], recv_sem=recv_sems.at[s, q],
                device_id=(peer[d],), device_id_type=pl.DeviceIdType.MESH,
            )
            rdma.start()
            return rdma

        def ag_chunk(t, d):
            return me + 1 - t if d == 0 else me - 1 + t

        def ag_send(t, d, q):
            send_sems, recv_sems = ag_sems[d]
            k = h_slot(ag_chunk(t, d), d, q)
            rdma = pltpu.make_async_remote_copy(
                src_ref=h_ref.at[k], dst_ref=h_ref.at[k],
                send_sem=send_sems.at[t, q], recv_sem=recv_sems.at[t, q],
                device_id=(peer[d],), device_id_type=pl.DeviceIdType.MESH,
            )
            rdma.start()
            return rdma

        def rs_recv_chunk(s, d):
            return me - 1 - s if d == 0 else me + 1 + s

        barrier = pltpu.get_barrier_semaphore()
        for nbr in (left, right):
            pl.semaphore_signal(barrier, inc=1, device_id=(nbr,),
                                device_id_type=pl.DeviceIdType.MESH)

        sends = {}

        for q in range(NQ):
            for d in (0, 1):
                gemm1_piece(me, d, q)
            if q == 0:
                pl.semaphore_wait(barrier, 2)
            for d in (0, 1):
                sends[("rs", 0, d, q)] = rs_send(0, d, q)
        gemm1(me - 1)
        gemm1(me + 1)
        if not COMM_ONLY:
            w2b_ref[...] = w2_ref[...].astype(jnp.bfloat16)

        for s in (1, 2):
            for d, q in STREAMS:
                sends[("rs", s - 1, d, q)].wait_recv()
                reg = rs_region(s - 1, d, q)
                rs_buf[reg, :] = (
                    rs_buf[reg, :]
                    + h_ref[h_slot(rs_recv_chunk(s - 1, d), d, q)])
                sends[("rs", s, d, q)] = rs_send(s, d, q)
            if s == 1:
                gemm1(me + 2)

        for d, q in STREAMS:
            sends[("rs", 2, d, q)].wait_recv()
            k = h_slot(me + 1 if d == 0 else me - 1, d, q)
            h_ref[k] = rs_buf[rs_region(2, d, q), :] + h_ref[k]
            sends[("ag", 0, d, q)] = ag_send(0, d, q)

        for s in (0, 1, 2):
            for d, q in STREAMS:
                sends[("rs", s, d, q)].wait_send()

        for t in (1, 2):
            for d, q in STREAMS:
                sends[("ag", t - 1, d, q)].wait_recv()
                sends[("ag", t, d, q)] = ag_send(t, d, q)
            if t == 1:
                gemm2(me)
            else:
                gemm2(me + 1)
                gemm2(me - 1)
        for d, q in STREAMS:
            sends[("ag", 2, d, q)].wait_recv()
            gemm2_piece(me + 2, d, q)
        for t in (0, 1, 2):
            for d, q in STREAMS:
                sends[("ag", t, d, q)].wait_send()

        @functools.partial(pl.run_scoped, sem=pltpu.SemaphoreType.REGULAR)
        def _(sem):
            for nbr in (left, right):
                pl.semaphore_signal(sem, inc=1, device_id=(nbr,),
                                    device_id_type=pl.DeviceIdType.MESH)
            pl.semaphore_wait(sem, 2)

    return pl.pallas_call(
        body,
        out_shape=jax.ShapeDtypeStruct((M, N), jnp.float32),
        in_specs=[pl.BlockSpec(memory_space=pltpu.VMEM)] * 3,
        out_specs=pl.BlockSpec(memory_space=pltpu.VMEM),
        scratch_shapes=[
            pltpu.VMEM((2 * NQ * N_DEV, QR, D), jnp.bfloat16),
            pltpu.VMEM(((N_DEV - 1) * CH, D), jnp.bfloat16),
            pltpu.VMEM((D, N), jnp.bfloat16),
            pltpu.SemaphoreType.DMA((N_DEV - 1, NQ)),
            pltpu.SemaphoreType.DMA((N_DEV - 1, NQ)),
            pltpu.SemaphoreType.DMA((N_DEV - 1, NQ)),
            pltpu.SemaphoreType.DMA((N_DEV - 1, NQ)),
            pltpu.SemaphoreType.DMA((N_DEV - 1, NQ)),
            pltpu.SemaphoreType.DMA((N_DEV - 1, NQ)),
            pltpu.SemaphoreType.DMA((N_DEV - 1, NQ)),
            pltpu.SemaphoreType.DMA((N_DEV - 1, NQ)),
        ],
        compiler_params=pltpu.CompilerParams(
            collective_id=0, vmem_limit_bytes=63 * 1024 * 1024),
    )(x, W1, W2)
